# initial kernel scaffold (unmeasured)
import jax
import jax.numpy as jnp
from jax import lax
from jax.experimental import pallas as pl
from jax.experimental.pallas import tpu as pltpu

K = 2048
D = 2048
F = 8192
M_HALF = D // 2
NCHUNK = 8
FC = F // NCHUNK


def _gemm_body(x_ref, dy_ref, mine_ref, send_ref):
    my_y = lax.axis_index("y")
    xb = x_ref[:].astype(jnp.bfloat16)
    dyb = dy_ref[:].astype(jnp.bfloat16)
    part = lax.dot_general(
        xb, dyb, (((0,), (0,)), ((), ())),
        preferred_element_type=jnp.float32,
    )
    mine = lax.dynamic_slice_in_dim(part, my_y * M_HALF, M_HALF, 0)
    theirs = lax.dynamic_slice_in_dim(part, (1 - my_y) * M_HALF, M_HALF, 0)
    mine_ref[:] = mine
    send_ref[:] = theirs.astype(jnp.bfloat16)


def _comm_body(send_ref, recv_ref, send_sem, recv_sem):
    my_x = lax.axis_index("x")
    my_y = lax.axis_index("y")
    my_z = lax.axis_index("z")
    partner = (my_x, 1 - my_y, my_z)

    barrier = pltpu.get_barrier_semaphore()
    pl.semaphore_signal(
        barrier, inc=1, device_id=partner,
        device_id_type=pl.DeviceIdType.MESH,
    )
    pl.semaphore_wait(barrier, 1)

    rdma = pltpu.make_async_remote_copy(
        src_ref=send_ref,
        dst_ref=recv_ref,
        send_sem=send_sem,
        recv_sem=recv_sem,
        device_id=partner,
        device_id_type=pl.DeviceIdType.MESH,
    )
    rdma.start()
    rdma.wait()


def _add_body(mine_ref, recv_ref, out_ref):
    out_ref[:] = mine_ref[:] + recv_ref[:].astype(jnp.float32)


def kernel(x, dy):
    mine, send = pl.pallas_call(
        _gemm_body,
        grid=(NCHUNK,),
        in_specs=[
            pl.BlockSpec((K, D), lambda j: (0, 0)),
            pl.BlockSpec((K, FC), lambda j: (0, j)),
        ],
        out_specs=[
            pl.BlockSpec((M_HALF, FC), lambda j: (0, j)),
            pl.BlockSpec((M_HALF, FC), lambda j: (0, j)),
        ],
        out_shape=[
            jax.ShapeDtypeStruct((M_HALF, F), jnp.float32),
            jax.ShapeDtypeStruct((M_HALF, F), jnp.bfloat16),
        ],
    )(x, dy)

    recv = pl.pallas_call(
        _comm_body,
        in_specs=[pl.BlockSpec(memory_space=pltpu.ANY)],
        out_specs=pl.BlockSpec(memory_space=pltpu.ANY),
        out_shape=jax.ShapeDtypeStruct((M_HALF, F), jnp.bfloat16),
        scratch_shapes=[
            pltpu.SemaphoreType.DMA,
            pltpu.SemaphoreType.DMA,
        ],
        compiler_params=pltpu.CompilerParams(collective_id=0),
    )(send)

    out = pl.pallas_call(
        _add_body,
        grid=(NCHUNK,),
        in_specs=[
            pl.BlockSpec((M_HALF, FC), lambda j: (0, j)),
            pl.BlockSpec((M_HALF, FC), lambda j: (0, j)),
        ],
        out_specs=pl.BlockSpec((M_HALF, FC), lambda j: (0, j)),
        out_shape=jax.ShapeDtypeStruct((M_HALF, F), jnp.float32),
    )(mine, recv)
    return out


# baseline (device time: 306166 ns/iter reference)
import jax
import jax.numpy as jnp
from jax import lax
from jax.experimental import pallas as pl
from jax.experimental.pallas import tpu as pltpu

K = 2048
D = 2048
F = 8192
M_HALF = D // 2
NCHUNK = 8
FC = F // NCHUNK


def _gemm_body(x_ref, dy_ref, mine_ref, send_ref):
    my_y = lax.axis_index("y")
    xb = x_ref[:].astype(jnp.bfloat16)
    dyb = dy_ref[:].astype(jnp.bfloat16)
    part = lax.dot_general(
        xb, dyb, (((0,), (0,)), ((), ())),
        preferred_element_type=jnp.float32,
    )
    top = part[:M_HALF]
    bot = part[M_HALF:]

    @pl.when(my_y == 0)
    def _():
        mine_ref[:] = top
        send_ref[:] = bot.astype(jnp.bfloat16)

    @pl.when(my_y == 1)
    def _():
        mine_ref[:] = bot
        send_ref[:] = top.astype(jnp.bfloat16)


def _comm_body(send_ref, recv_ref, send_sem, recv_sem):
    my_x = lax.axis_index("x")
    my_y = lax.axis_index("y")
    my_z = lax.axis_index("z")
    partner = (my_x, 1 - my_y, my_z)

    barrier = pltpu.get_barrier_semaphore()
    pl.semaphore_signal(
        barrier, inc=1, device_id=partner,
        device_id_type=pl.DeviceIdType.MESH,
    )
    pl.semaphore_wait(barrier, 1)

    rdma = pltpu.make_async_remote_copy(
        src_ref=send_ref,
        dst_ref=recv_ref,
        send_sem=send_sem,
        recv_sem=recv_sem,
        device_id=partner,
        device_id_type=pl.DeviceIdType.MESH,
    )
    rdma.start()
    rdma.wait()


def _add_body(mine_ref, recv_ref, out_ref):
    out_ref[:] = mine_ref[:] + recv_ref[:].astype(jnp.float32)


def kernel(x, dy):
    mine, send = pl.pallas_call(
        _gemm_body,
        grid=(NCHUNK,),
        in_specs=[
            pl.BlockSpec((K, D), lambda j: (0, 0)),
            pl.BlockSpec((K, FC), lambda j: (0, j)),
        ],
        out_specs=[
            pl.BlockSpec((M_HALF, FC), lambda j: (0, j)),
            pl.BlockSpec((M_HALF, FC), lambda j: (0, j)),
        ],
        out_shape=[
            jax.ShapeDtypeStruct((M_HALF, F), jnp.float32),
            jax.ShapeDtypeStruct((M_HALF, F), jnp.bfloat16),
        ],
        compiler_params=pltpu.CompilerParams(
            vmem_limit_bytes=100 * 1024 * 1024,
        ),
    )(x, dy)

    recv = pl.pallas_call(
        _comm_body,
        in_specs=[pl.BlockSpec(memory_space=pl.ANY)],
        out_specs=pl.BlockSpec(memory_space=pl.ANY),
        out_shape=jax.ShapeDtypeStruct((M_HALF, F), jnp.bfloat16),
        scratch_shapes=[
            pltpu.SemaphoreType.DMA,
            pltpu.SemaphoreType.DMA,
        ],
        compiler_params=pltpu.CompilerParams(collective_id=0),
    )(send)

    out = pl.pallas_call(
        _add_body,
        grid=(NCHUNK,),
        in_specs=[
            pl.BlockSpec((M_HALF, FC), lambda j: (0, j)),
            pl.BlockSpec((M_HALF, FC), lambda j: (0, j)),
        ],
        out_specs=pl.BlockSpec((M_HALF, FC), lambda j: (0, j)),
        out_shape=jax.ShapeDtypeStruct((M_HALF, F), jnp.float32),
        compiler_params=pltpu.CompilerParams(
            vmem_limit_bytes=100 * 1024 * 1024,
        ),
    )(mine, recv)
    return out


# device time: 165476 ns/iter; 1.8502x vs baseline; 1.8502x over previous
import jax
import jax.numpy as jnp
from jax import lax
from jax.experimental import pallas as pl
from jax.experimental.pallas import tpu as pltpu

K = 2048
D = 2048
F = 8192
M_HALF = D // 2
NRING = 8
FC = F // NRING
CW_HOPS = 4
CCW_HOPS = 3


def _ring_coords(q):
    qx = (q >= 4).astype(jnp.int32)
    qz = jnp.where(qx == 0, q, 7 - q)
    return qx, qz


def _body(x_hbm, dy_hbm, out_hbm,
          x_v, dy_v, y_send, y_recv, red_bf, cw_buf, ccw_buf,
          load_sems, store_sems, y_sems, cw_send, cw_recv, ccw_send,
          ccw_recv):
    my_x = lax.axis_index("x")
    my_y = lax.axis_index("y")
    my_z = lax.axis_index("z")
    p = jnp.where(my_x == 0, my_z, 7 - my_z)

    rx, rz = _ring_coords((p + 1) % NRING)
    lx, lz = _ring_coords((p - 1) % NRING)
    partner = (my_x, 1 - my_y, my_z)
    right = (rx, my_y, rz)
    left = (lx, my_y, lz)

    xcopy = pltpu.make_async_copy(x_hbm, x_v, load_sems.at[0])
    xcopy.start()
    dycopy = pltpu.make_async_copy(
        dy_hbm.at[:, pl.ds(p * FC, FC)], dy_v, load_sems.at[1]
    )
    dycopy.start()

    barrier = pltpu.get_barrier_semaphore()
    for nbr in (partner, left, right):
        pl.semaphore_signal(
            barrier, inc=1, device_id=nbr,
            device_id_type=pl.DeviceIdType.MESH,
        )
    pl.semaphore_wait(barrier, 3)

    xcopy.wait()
    dycopy.wait()

    part = lax.dot_general(
        x_v[:], dy_v[:].astype(jnp.bfloat16),
        (((0,), (0,)), ((), ())),
        preferred_element_type=jnp.float32,
    )
    top = part[:M_HALF]
    bot = part[M_HALF:]

    @pl.when(my_y == 0)
    def _():
        y_send[:] = bot.astype(jnp.bfloat16)

    @pl.when(my_y == 1)
    def _():
        y_send[:] = top.astype(jnp.bfloat16)

    y_rdma = pltpu.make_async_remote_copy(
        src_ref=y_send, dst_ref=y_recv,
        send_sem=y_sems.at[0], recv_sem=y_sems.at[1],
        device_id=partner, device_id_type=pl.DeviceIdType.MESH,
    )
    y_rdma.start()
    y_rdma.wait()

    @pl.when(my_y == 0)
    def _():
        red_bf[:] = (top + y_recv[:].astype(jnp.float32)).astype(jnp.bfloat16)

    @pl.when(my_y == 1)
    def _():
        red_bf[:] = (bot + y_recv[:].astype(jnp.float32)).astype(jnp.bfloat16)

    store_jobs = []

    def store_chunk(origin, src_ref):
        si = len(store_jobs)
        if si >= 2:
            store_jobs[si - 2].wait()
        st = pltpu.make_async_copy(
            src_ref,
            out_hbm.at[:, pl.ds(origin * FC, FC)],
            store_sems.at[si % 2],
        )
        st.start()
        store_jobs.append(st)

    store_chunk(p, red_bf)

    def cw_rdma(h):
        src = red_bf if h == 0 else cw_buf.at[h - 1]
        return pltpu.make_async_remote_copy(
            src_ref=src, dst_ref=cw_buf.at[h],
            send_sem=cw_send.at[h], recv_sem=cw_recv.at[h],
            device_id=right, device_id_type=pl.DeviceIdType.MESH,
        )

    def ccw_rdma(h):
        src = red_bf if h == 0 else ccw_buf.at[h - 1]
        return pltpu.make_async_remote_copy(
            src_ref=src, dst_ref=ccw_buf.at[h],
            send_sem=ccw_send.at[h], recv_sem=ccw_recv.at[h],
            device_id=left, device_id_type=pl.DeviceIdType.MESH,
        )

    rdmas = []
    cw_prev = cw_rdma(0)
    cw_prev.start()
    rdmas.append(cw_prev)
    ccw_prev = ccw_rdma(0)
    ccw_prev.start()
    rdmas.append(ccw_prev)

    for h in range(CW_HOPS):
        cw_prev.wait_recv()
        if h + 1 < CW_HOPS:
            cw_prev = cw_rdma(h + 1)
            cw_prev.start()
            rdmas.append(cw_prev)
        store_chunk((p - h - 1) % NRING, cw_buf.at[h])

        if h < CCW_HOPS:
            ccw_prev.wait_recv()
            if h + 1 < CCW_HOPS:
                ccw_prev = ccw_rdma(h + 1)
                ccw_prev.start()
                rdmas.append(ccw_prev)
            store_chunk((p + h + 1) % NRING, ccw_buf.at[h])

    for r in rdmas:
        r.wait_send()
    for st in store_jobs[-2:]:
        st.wait()


def kernel(x, dy):
    return pl.pallas_call(
        _body,
        in_specs=[
            pl.BlockSpec(memory_space=pl.ANY),
            pl.BlockSpec(memory_space=pl.ANY),
        ],
        out_specs=pl.BlockSpec(memory_space=pl.ANY),
        out_shape=jax.ShapeDtypeStruct((M_HALF, F), jnp.bfloat16),
        scratch_shapes=[
            pltpu.VMEM((K, D), jnp.bfloat16),
            pltpu.VMEM((K, FC), jnp.float32),
            pltpu.VMEM((M_HALF, FC), jnp.bfloat16),
            pltpu.VMEM((M_HALF, FC), jnp.bfloat16),
            pltpu.VMEM((M_HALF, FC), jnp.bfloat16),
            pltpu.VMEM((CW_HOPS, M_HALF, FC), jnp.bfloat16),
            pltpu.VMEM((CCW_HOPS, M_HALF, FC), jnp.bfloat16),
            pltpu.SemaphoreType.DMA((2,)),
            pltpu.SemaphoreType.DMA((2,)),
            pltpu.SemaphoreType.DMA((2,)),
            pltpu.SemaphoreType.DMA((CW_HOPS,)),
            pltpu.SemaphoreType.DMA((CW_HOPS,)),
            pltpu.SemaphoreType.DMA((CCW_HOPS,)),
            pltpu.SemaphoreType.DMA((CCW_HOPS,)),
        ],
        compiler_params=pltpu.CompilerParams(
            collective_id=0,
            vmem_limit_bytes=60 * 1024 * 1024,
        ),
    )(x.astype(jnp.bfloat16), dy)


# device time: 133437 ns/iter; 2.2945x vs baseline; 1.2401x over previous
import jax
import jax.numpy as jnp
from jax import lax
from jax.experimental import pallas as pl
from jax.experimental.pallas import tpu as pltpu

K = 2048
D = 2048
F = 8192
M_HALF = D // 2
NRING = 8
FC = F // NRING
SC = FC // 2
LONG = 4
SHORT = 3


def _ring_coords(q):
    qx = (q >= 4).astype(jnp.int32)
    qz = jnp.where(qx == 0, q, 7 - q)
    return qx, qz


def _body(x_hbm, dy_hbm, out_hbm,
          x_v, dy_v, y_send, y_recv, red_bf,
          cw_long, cw_short, ccw_long, ccw_short,
          load_sems, store_sems, y_ssem, y_rsem,
          cwl_s, cwl_r, cws_s, cws_r, ccwl_s, ccwl_r, ccws_s, ccws_r):
    my_x = lax.axis_index("x")
    my_y = lax.axis_index("y")
    my_z = lax.axis_index("z")
    p = jnp.where(my_x == 0, my_z, 7 - my_z)

    rx, rz = _ring_coords((p + 1) % NRING)
    lx, lz = _ring_coords((p - 1) % NRING)
    partner = (my_x, 1 - my_y, my_z)
    right = (rx, my_y, rz)
    left = (lx, my_y, lz)

    xcopy = pltpu.make_async_copy(x_hbm, x_v, load_sems.at[0])
    xcopy.start()
    dycopy = pltpu.make_async_copy(
        dy_hbm.at[:, pl.ds(p * FC, FC)], dy_v, load_sems.at[1]
    )
    dycopy.start()

    barrier = pltpu.get_barrier_semaphore()
    for nbr in (partner, left, right):
        pl.semaphore_signal(
            barrier, inc=1, device_id=nbr,
            device_id_type=pl.DeviceIdType.MESH,
        )
    pl.semaphore_wait(barrier, 3)

    xcopy.wait()
    dycopy.wait()

    y_rdmas = []
    parts = []
    for s in range(2):
        part = lax.dot_general(
            x_v[:], dy_v[:, s * SC:(s + 1) * SC].astype(jnp.bfloat16),
            (((0,), (0,)), ((), ())),
            preferred_element_type=jnp.float32,
        )
        parts.append(part)

        @pl.when(my_y == 0)
        def _():
            y_send[s, :, :] = part[M_HALF:].astype(jnp.bfloat16)

        @pl.when(my_y == 1)
        def _():
            y_send[s, :, :] = part[:M_HALF].astype(jnp.bfloat16)

        r = pltpu.make_async_remote_copy(
            src_ref=y_send.at[s], dst_ref=y_recv.at[s],
            send_sem=y_ssem.at[s], recv_sem=y_rsem.at[s],
            device_id=partner, device_id_type=pl.DeviceIdType.MESH,
        )
        r.start()
        y_rdmas.append(r)

    store_jobs = []

    def store_sub(origin, sub, src_ref):
        si = len(store_jobs)
        if si >= 4:
            store_jobs[si - 4].wait()
        st = pltpu.make_async_copy(
            src_ref,
            out_hbm.at[:, pl.ds(origin * FC + sub * SC, SC)],
            store_sems.at[si % 4],
        )
        st.start()
        store_jobs.append(st)

    chains = {
        "cwl": (cw_long, cwl_s, cwl_r, right, LONG),
        "ccws": (ccw_short, ccws_s, ccws_r, left, SHORT),
        "ccwl": (ccw_long, ccwl_s, ccwl_r, left, LONG),
        "cws": (cw_short, cws_s, cws_r, right, SHORT),
    }

    def hop_rdma(name, h, sub):
        buf, ssem, rsem, tgt, _hops = chains[name]
        src = red_bf.at[sub] if h == 0 else buf.at[h - 1]
        return pltpu.make_async_remote_copy(
            src_ref=src, dst_ref=buf.at[h],
            send_sem=ssem.at[h], recv_sem=rsem.at[h],
            device_id=tgt, device_id_type=pl.DeviceIdType.MESH,
        )

    rdmas = []
    live = {}
    for s in range(2):
        y_rdmas[s].wait()

        @pl.when(my_y == 0)
        def _():
            red_bf[s, :, :] = (
                parts[s][:M_HALF] + y_recv[s].astype(jnp.float32)
            ).astype(jnp.bfloat16)

        @pl.when(my_y == 1)
        def _():
            red_bf[s, :, :] = (
                parts[s][M_HALF:] + y_recv[s].astype(jnp.float32)
            ).astype(jnp.bfloat16)

        store_sub(p, s, red_bf.at[s])
        for name in (("cwl", "ccws") if s == 0 else ("ccwl", "cws")):
            r = hop_rdma(name, 0, s)
            r.start()
            rdmas.append(r)
            live[name] = r

    for h in range(LONG):
        order = ["cwl", "ccwl"] + (["cws", "ccws"] if h < SHORT else [])
        for name in order:
            buf, _ssem, _rsem, tgt, hops = chains[name]
            sub = 0 if name in ("cwl", "ccws") else 1
            live[name].wait_recv()
            if h + 1 < hops:
                r = hop_rdma(name, h + 1, sub)
                r.start()
                rdmas.append(r)
                live[name] = r
            sign = -1 if name.startswith("cw") else 1
            store_sub((p + sign * (h + 1)) % NRING, sub, buf.at[h])

    for r in rdmas:
        r.wait_send()
    for st in store_jobs[-4:]:
        st.wait()


def kernel(x, dy):
    return pl.pallas_call(
        _body,
        in_specs=[
            pl.BlockSpec(memory_space=pl.ANY),
            pl.BlockSpec(memory_space=pl.ANY),
        ],
        out_specs=pl.BlockSpec(memory_space=pl.ANY),
        out_shape=jax.ShapeDtypeStruct((M_HALF, F), jnp.bfloat16),
        scratch_shapes=[
            pltpu.VMEM((K, D), jnp.bfloat16),
            pltpu.VMEM((K, FC), jnp.float32),
            pltpu.VMEM((2, M_HALF, SC), jnp.bfloat16),
            pltpu.VMEM((2, M_HALF, SC), jnp.bfloat16),
            pltpu.VMEM((2, M_HALF, SC), jnp.bfloat16),
            pltpu.VMEM((LONG, M_HALF, SC), jnp.bfloat16),
            pltpu.VMEM((SHORT, M_HALF, SC), jnp.bfloat16),
            pltpu.VMEM((LONG, M_HALF, SC), jnp.bfloat16),
            pltpu.VMEM((SHORT, M_HALF, SC), jnp.bfloat16),
            pltpu.SemaphoreType.DMA((2,)),
            pltpu.SemaphoreType.DMA((4,)),
            pltpu.SemaphoreType.DMA((2,)),
            pltpu.SemaphoreType.DMA((2,)),
            pltpu.SemaphoreType.DMA((LONG,)),
            pltpu.SemaphoreType.DMA((LONG,)),
            pltpu.SemaphoreType.DMA((SHORT,)),
            pltpu.SemaphoreType.DMA((SHORT,)),
            pltpu.SemaphoreType.DMA((LONG,)),
            pltpu.SemaphoreType.DMA((LONG,)),
            pltpu.SemaphoreType.DMA((SHORT,)),
            pltpu.SemaphoreType.DMA((SHORT,)),
        ],
        compiler_params=pltpu.CompilerParams(
            collective_id=0,
            vmem_limit_bytes=60 * 1024 * 1024,
        ),
    )(x.astype(jnp.bfloat16), dy)


# device time: 122966 ns/iter; 2.4898x vs baseline; 1.0852x over previous
import jax
import jax.numpy as jnp
from jax import lax
from jax.experimental import pallas as pl
from jax.experimental.pallas import tpu as pltpu

K = 2048
D = 2048
F = 8192
M_HALF = D // 2
NRING = 8
FC = F // NRING
NSUB = 4
SC = FC // NSUB
HOPS = 3
P4SC = FC // 2


def _ring_coords(q):
    qx = (q >= 4).astype(jnp.int32)
    qz = jnp.where(qx == 0, q, 7 - q)
    return qx, qz


def _body(x_hbm, dy_hbm, out_hbm,
          x_v, dy_v, y_send, y_recv, red_bf, cw_buf, ccw_buf,
          y4_send, y4_recv, red4_bf,
          load_sems, store_sems, y_ssem, y_rsem, y4_ssem, y4_rsem,
          cw_ssem, cw_rsem, ccw_ssem, ccw_rsem):
    my_x = lax.axis_index("x")
    my_y = lax.axis_index("y")
    my_z = lax.axis_index("z")
    p = jnp.where(my_x == 0, my_z, 7 - my_z)
    p4 = (p + 4) % NRING

    rx, rz = _ring_coords((p + 1) % NRING)
    lx, lz = _ring_coords((p - 1) % NRING)
    partner = (my_x, 1 - my_y, my_z)
    right = (rx, my_y, rz)
    left = (lx, my_y, lz)

    xcopy = pltpu.make_async_copy(x_hbm, x_v, load_sems.at[0])
    xcopy.start()
    dycopy = pltpu.make_async_copy(
        dy_hbm.at[:, pl.ds(p * FC, FC)], dy_v, load_sems.at[1]
    )
    dycopy.start()

    barrier = pltpu.get_barrier_semaphore()
    for nbr in (partner, left, right):
        pl.semaphore_signal(
            barrier, inc=1, device_id=nbr,
            device_id_type=pl.DeviceIdType.MESH,
        )
    pl.semaphore_wait(barrier, 3)

    xcopy.wait()
    dycopy.wait()

    y_rdmas = []
    mines = []
    for s in range(NSUB):
        part = lax.dot_general(
            x_v[:], dy_v[:, s * SC:(s + 1) * SC].astype(jnp.bfloat16),
            (((0,), (0,)), ((), ())),
            preferred_element_type=jnp.float32,
        )

        @pl.when(my_y == 0)
        def _():
            y_send[s, :, :] = part[M_HALF:].astype(jnp.bfloat16)

        @pl.when(my_y == 1)
        def _():
            y_send[s, :, :] = part[:M_HALF].astype(jnp.bfloat16)

        mines.append(part)

        r = pltpu.make_async_remote_copy(
            src_ref=y_send.at[s], dst_ref=y_recv.at[s],
            send_sem=y_ssem.at[s], recv_sem=y_rsem.at[s],
            device_id=partner, device_id_type=pl.DeviceIdType.MESH,
        )
        r.start()
        y_rdmas.append(r)

    dy4copy = pltpu.make_async_copy(
        dy_hbm.at[:, pl.ds(p4 * FC, FC)], dy_v, load_sems.at[1]
    )
    dy4copy.start()

    store_jobs = []

    def store(col_start, width, src_ref):
        si = len(store_jobs)
        if si >= 4:
            store_jobs[si - 4].wait()
        st = pltpu.make_async_copy(
            src_ref,
            out_hbm.at[:, pl.ds(col_start, width)],
            store_sems.at[si % 4],
        )
        st.start()
        store_jobs.append(st)

    def hop_rdma(dirn, s, h):
        buf = cw_buf if dirn == "cw" else ccw_buf
        ssem = cw_ssem if dirn == "cw" else ccw_ssem
        rsem = cw_rsem if dirn == "cw" else ccw_rsem
        tgt = right if dirn == "cw" else left
        src = red_bf.at[s] if h == 0 else buf.at[s, h - 1]
        return pltpu.make_async_remote_copy(
            src_ref=src, dst_ref=buf.at[s, h],
            send_sem=ssem.at[s * HOPS + h], recv_sem=rsem.at[s * HOPS + h],
            device_id=tgt, device_id_type=pl.DeviceIdType.MESH,
        )

    rdmas = []
    live = {}
    for s in range(NSUB):
        y_rdmas[s].wait()

        @pl.when(my_y == 0)
        def _():
            red_bf[s, :, :] = (
                mines[s][:M_HALF] + y_recv[s].astype(jnp.float32)
            ).astype(jnp.bfloat16)

        @pl.when(my_y == 1)
        def _():
            red_bf[s, :, :] = (
                mines[s][M_HALF:] + y_recv[s].astype(jnp.float32)
            ).astype(jnp.bfloat16)

        for dirn in ("cw", "ccw"):
            r = hop_rdma(dirn, s, 0)
            r.start()
            rdmas.append(r)
            live[(dirn, s)] = r
        store(p * FC + s * SC, SC, red_bf.at[s])

    y4_rdmas = []
    mines4 = []
    for h in range(HOPS):
        for s in range(NSUB):
            for dirn in ("cw", "ccw"):
                live[(dirn, s)].wait_recv()
                if h + 1 < HOPS:
                    r = hop_rdma(dirn, s, h + 1)
                    r.start()
                    rdmas.append(r)
                    live[(dirn, s)] = r

        if h < 2:
            if h == 0:
                dy4copy.wait()
            part4 = lax.dot_general(
                x_v[:],
                dy_v[:, h * P4SC:(h + 1) * P4SC].astype(jnp.bfloat16),
                (((0,), (0,)), ((), ())),
                preferred_element_type=jnp.float32,
            )

            @pl.when(my_y == 0)
            def _():
                y4_send[h, :, :] = part4[M_HALF:].astype(jnp.bfloat16)

            @pl.when(my_y == 1)
            def _():
                y4_send[h, :, :] = part4[:M_HALF].astype(jnp.bfloat16)

            mines4.append(part4)
            r4 = pltpu.make_async_remote_copy(
                src_ref=y4_send.at[h], dst_ref=y4_recv.at[h],
                send_sem=y4_ssem.at[h], recv_sem=y4_rsem.at[h],
                device_id=partner, device_id_type=pl.DeviceIdType.MESH,
            )
            r4.start()
            y4_rdmas.append(r4)

        for s in range(NSUB):
            store(((p - h - 1) % NRING) * FC + s * SC, SC, cw_buf.at[s, h])
            store(((p + h + 1) % NRING) * FC + s * SC, SC, ccw_buf.at[s, h])

    for h in range(2):
        y4_rdmas[h].wait()

        @pl.when(my_y == 0)
        def _():
            red4_bf[h, :, :] = (
                mines4[h][:M_HALF] + y4_recv[h].astype(jnp.float32)
            ).astype(jnp.bfloat16)

        @pl.when(my_y == 1)
        def _():
            red4_bf[h, :, :] = (
                mines4[h][M_HALF:] + y4_recv[h].astype(jnp.float32)
            ).astype(jnp.bfloat16)

        store(p4 * FC + h * P4SC, P4SC, red4_bf.at[h])

    for r in rdmas:
        r.wait_send()
    for st in store_jobs[-4:]:
        st.wait()


def kernel(x, dy):
    return pl.pallas_call(
        _body,
        in_specs=[
            pl.BlockSpec(memory_space=pl.ANY),
            pl.BlockSpec(memory_space=pl.ANY),
        ],
        out_specs=pl.BlockSpec(memory_space=pl.ANY),
        out_shape=jax.ShapeDtypeStruct((M_HALF, F), jnp.bfloat16),
        scratch_shapes=[
            pltpu.VMEM((K, D), jnp.bfloat16),
            pltpu.VMEM((K, FC), jnp.float32),
            pltpu.VMEM((NSUB, M_HALF, SC), jnp.bfloat16),
            pltpu.VMEM((NSUB, M_HALF, SC), jnp.bfloat16),
            pltpu.VMEM((NSUB, M_HALF, SC), jnp.bfloat16),
            pltpu.VMEM((NSUB, HOPS, M_HALF, SC), jnp.bfloat16),
            pltpu.VMEM((NSUB, HOPS, M_HALF, SC), jnp.bfloat16),
            pltpu.VMEM((2, M_HALF, P4SC), jnp.bfloat16),
            pltpu.VMEM((2, M_HALF, P4SC), jnp.bfloat16),
            pltpu.VMEM((2, M_HALF, P4SC), jnp.bfloat16),
            pltpu.SemaphoreType.DMA((2,)),
            pltpu.SemaphoreType.DMA((4,)),
            pltpu.SemaphoreType.DMA((NSUB,)),
            pltpu.SemaphoreType.DMA((NSUB,)),
            pltpu.SemaphoreType.DMA((2,)),
            pltpu.SemaphoreType.DMA((2,)),
            pltpu.SemaphoreType.DMA((NSUB * HOPS,)),
            pltpu.SemaphoreType.DMA((NSUB * HOPS,)),
            pltpu.SemaphoreType.DMA((NSUB * HOPS,)),
            pltpu.SemaphoreType.DMA((NSUB * HOPS,)),
        ],
        compiler_params=pltpu.CompilerParams(
            collective_id=0,
            vmem_limit_bytes=62 * 1024 * 1024,
        ),
    )(x.astype(jnp.bfloat16), dy)


# device time: 115927 ns/iter; 2.6410x vs baseline; 1.0607x over previous
import jax
import jax.numpy as jnp
from jax import lax
from jax.experimental import pallas as pl
from jax.experimental.pallas import tpu as pltpu

K = 2048
D = 2048
F = 8192
M_HALF = D // 2
NRING = 8
FC = F // NRING
NSUB = 4
SC = FC // NSUB
HOPS = 3
P4SC = FC // 2
KH = K // 2


def _ring_coords(q):
    qx = (q >= 4).astype(jnp.int32)
    qz = jnp.where(qx == 0, q, 7 - q)
    return qx, qz


def _body(x_hbm, dy_hbm, out_hbm,
          x_v, x_stage, dy_v, y_send, y_recv, red_bf, cw_buf, ccw_buf,
          y4_send, y4_recv, red4_bf,
          load_sems, store_sems, y_ssem, y_rsem, y4_ssem, y4_rsem,
          cw_ssem, cw_rsem, ccw_ssem, ccw_rsem):
    my_x = lax.axis_index("x")
    my_y = lax.axis_index("y")
    my_z = lax.axis_index("z")
    p = jnp.where(my_x == 0, my_z, 7 - my_z)
    p4 = (p + 4) % NRING

    rx, rz = _ring_coords((p + 1) % NRING)
    lx, lz = _ring_coords((p - 1) % NRING)
    partner = (my_x, 1 - my_y, my_z)
    right = (rx, my_y, rz)
    left = (lx, my_y, lz)

    xcopies = []
    for i in range(2):
        c = pltpu.make_async_copy(
            x_hbm.at[pl.ds(i * KH, KH)], x_stage.at[i], load_sems.at[i]
        )
        c.start()
        xcopies.append(c)
    dycopy = pltpu.make_async_copy(
        dy_hbm.at[:, pl.ds(p * FC, FC)], dy_v, load_sems.at[2]
    )
    dycopy.start()

    barrier = pltpu.get_barrier_semaphore()
    for nbr in (partner, left, right):
        pl.semaphore_signal(
            barrier, inc=1, device_id=nbr,
            device_id_type=pl.DeviceIdType.MESH,
        )
    pl.semaphore_wait(barrier, 3)

    for i in range(2):
        xcopies[i].wait()
        x_v[i * KH:(i + 1) * KH, :] = x_stage[i].astype(jnp.bfloat16)
    dycopy.wait()

    y_rdmas = []
    for s in range(NSUB):
        part = lax.dot_general(
            x_v[:], dy_v[:, s * SC:(s + 1) * SC].astype(jnp.bfloat16),
            (((0,), (0,)), ((), ())),
            preferred_element_type=jnp.float32,
        )

        @pl.when(my_y == 0)
        def _():
            y_send[s, :, :] = part[M_HALF:].astype(jnp.bfloat16)
            red_bf[s, :, :] = part[:M_HALF].astype(jnp.bfloat16)

        @pl.when(my_y == 1)
        def _():
            y_send[s, :, :] = part[:M_HALF].astype(jnp.bfloat16)
            red_bf[s, :, :] = part[M_HALF:].astype(jnp.bfloat16)

        r = pltpu.make_async_remote_copy(
            src_ref=y_send.at[s], dst_ref=y_recv.at[s],
            send_sem=y_ssem.at[s], recv_sem=y_rsem.at[s],
            device_id=partner, device_id_type=pl.DeviceIdType.MESH,
        )
        r.start()
        y_rdmas.append(r)

    dy4copy = pltpu.make_async_copy(
        dy_hbm.at[:, pl.ds(p4 * FC, FC)], dy_v, load_sems.at[2]
    )
    dy4copy.start()

    store_jobs = []

    def store(col_start, width, src_ref):
        si = len(store_jobs)
        if si >= 4:
            store_jobs[si - 4].wait()
        st = pltpu.make_async_copy(
            src_ref,
            out_hbm.at[:, pl.ds(col_start, width)],
            store_sems.at[si % 4],
        )
        st.start()
        store_jobs.append(st)

    def hop_rdma(dirn, s, h):
        buf = cw_buf if dirn == "cw" else ccw_buf
        ssem = cw_ssem if dirn == "cw" else ccw_ssem
        rsem = cw_rsem if dirn == "cw" else ccw_rsem
        tgt = right if dirn == "cw" else left
        src = red_bf.at[s] if h == 0 else buf.at[s, h - 1]
        return pltpu.make_async_remote_copy(
            src_ref=src, dst_ref=buf.at[s, h],
            send_sem=ssem.at[s * HOPS + h], recv_sem=rsem.at[s * HOPS + h],
            device_id=tgt, device_id_type=pl.DeviceIdType.MESH,
        )

    rdmas = []
    live = {}
    for s in range(NSUB):
        y_rdmas[s].wait()
        red_bf[s, :, :] = (
            red_bf[s].astype(jnp.float32) + y_recv[s].astype(jnp.float32)
        ).astype(jnp.bfloat16)
        for dirn in ("cw", "ccw"):
            r = hop_rdma(dirn, s, 0)
            r.start()
            rdmas.append(r)
            live[(dirn, s)] = r
        store(p * FC + s * SC, SC, red_bf.at[s])

    y4_rdmas = []
    for h in range(HOPS):
        for s in range(NSUB):
            for dirn in ("cw", "ccw"):
                live[(dirn, s)].wait_recv()
                if h + 1 < HOPS:
                    r = hop_rdma(dirn, s, h + 1)
                    r.start()
                    rdmas.append(r)
                    live[(dirn, s)] = r

        if h < 2:
            if h == 0:
                dy4copy.wait()
            part4 = lax.dot_general(
                x_v[:],
                dy_v[:, h * P4SC:(h + 1) * P4SC].astype(jnp.bfloat16),
                (((0,), (0,)), ((), ())),
                preferred_element_type=jnp.float32,
            )

            @pl.when(my_y == 0)
            def _():
                y4_send[h, :, :] = part4[M_HALF:].astype(jnp.bfloat16)
                red4_bf[h, :, :] = part4[:M_HALF].astype(jnp.bfloat16)

            @pl.when(my_y == 1)
            def _():
                y4_send[h, :, :] = part4[:M_HALF].astype(jnp.bfloat16)
                red4_bf[h, :, :] = part4[M_HALF:].astype(jnp.bfloat16)

            r4 = pltpu.make_async_remote_copy(
                src_ref=y4_send.at[h], dst_ref=y4_recv.at[h],
                send_sem=y4_ssem.at[h], recv_sem=y4_rsem.at[h],
                device_id=partner, device_id_type=pl.DeviceIdType.MESH,
            )
            r4.start()
            y4_rdmas.append(r4)

        for s in range(NSUB):
            store(((p - h - 1) % NRING) * FC + s * SC, SC, cw_buf.at[s, h])
            store(((p + h + 1) % NRING) * FC + s * SC, SC, ccw_buf.at[s, h])

    for h in range(2):
        y4_rdmas[h].wait()
        red4_bf[h, :, :] = (
            red4_bf[h].astype(jnp.float32) + y4_recv[h].astype(jnp.float32)
        ).astype(jnp.bfloat16)
        store(p4 * FC + h * P4SC, P4SC, red4_bf.at[h])

    for r in rdmas:
        r.wait_send()
    for st in store_jobs[-4:]:
        st.wait()


def kernel(x, dy):
    return pl.pallas_call(
        _body,
        in_specs=[
            pl.BlockSpec(memory_space=pl.ANY),
            pl.BlockSpec(memory_space=pl.ANY),
        ],
        out_specs=pl.BlockSpec(memory_space=pl.ANY),
        out_shape=jax.ShapeDtypeStruct((M_HALF, F), jnp.bfloat16),
        scratch_shapes=[
            pltpu.VMEM((K, D), jnp.bfloat16),
            pltpu.VMEM((2, KH, D), jnp.float32),
            pltpu.VMEM((K, FC), jnp.float32),
            pltpu.VMEM((NSUB, M_HALF, SC), jnp.bfloat16),
            pltpu.VMEM((NSUB, M_HALF, SC), jnp.bfloat16),
            pltpu.VMEM((NSUB, M_HALF, SC), jnp.bfloat16),
            pltpu.VMEM((NSUB, HOPS, M_HALF, SC), jnp.bfloat16),
            pltpu.VMEM((NSUB, HOPS, M_HALF, SC), jnp.bfloat16),
            pltpu.VMEM((2, M_HALF, P4SC), jnp.bfloat16),
            pltpu.VMEM((2, M_HALF, P4SC), jnp.bfloat16),
            pltpu.VMEM((2, M_HALF, P4SC), jnp.bfloat16),
            pltpu.SemaphoreType.DMA((3,)),
            pltpu.SemaphoreType.DMA((4,)),
            pltpu.SemaphoreType.DMA((NSUB,)),
            pltpu.SemaphoreType.DMA((NSUB,)),
            pltpu.SemaphoreType.DMA((2,)),
            pltpu.SemaphoreType.DMA((2,)),
            pltpu.SemaphoreType.DMA((NSUB * HOPS,)),
            pltpu.SemaphoreType.DMA((NSUB * HOPS,)),
            pltpu.SemaphoreType.DMA((NSUB * HOPS,)),
            pltpu.SemaphoreType.DMA((NSUB * HOPS,)),
        ],
        compiler_params=pltpu.CompilerParams(
            collective_id=0,
            vmem_limit_bytes=62 * 1024 * 1024,
        ),
    )(x, dy)


# device time: 109856 ns/iter; 2.7870x vs baseline; 1.0553x over previous
import jax
import jax.numpy as jnp
from jax import lax
from jax.experimental import pallas as pl
from jax.experimental.pallas import tpu as pltpu

K = 2048
D = 2048
F = 8192
M_HALF = D // 2
NRING = 8
FC = F // NRING
NSUB = 4
SC = FC // NSUB
HOPS = 3
P4SC = FC // 2
KH = K // 2


def _ring_coords(q):
    qx = (q >= 4).astype(jnp.int32)
    qz = jnp.where(qx == 0, q, 7 - q)
    return qx, qz


def _body(x_hbm, dy_hbm, out_hbm,
          x_p, x_m, x_stage, dy_v, y_send, y_recv, red_bf, cw_buf,
          ccw_buf, y4_send, y4_recv, red4_bf,
          load_sems, store_sems, y_ssem, y_rsem, y4_ssem, y4_rsem,
          cw_ssem, cw_rsem, ccw_ssem, ccw_rsem):
    my_x = lax.axis_index("x")
    my_y = lax.axis_index("y")
    my_z = lax.axis_index("z")
    p = jnp.where(my_x == 0, my_z, 7 - my_z)
    p4 = (p + 4) % NRING

    rx, rz = _ring_coords((p + 1) % NRING)
    lx, lz = _ring_coords((p - 1) % NRING)
    partner = (my_x, 1 - my_y, my_z)
    right = (rx, my_y, rz)
    left = (lx, my_y, lz)

    xcopies = []
    for i in range(2):
        c = pltpu.make_async_copy(
            x_hbm.at[pl.ds(i * KH, KH)], x_stage.at[i], load_sems.at[i]
        )
        c.start()
        xcopies.append(c)
    dycopy = pltpu.make_async_copy(
        dy_hbm.at[:, pl.ds(p * FC, FC)], dy_v, load_sems.at[2]
    )
    dycopy.start()

    barrier = pltpu.get_barrier_semaphore()
    for nbr in (partner, left, right):
        pl.semaphore_signal(
            barrier, inc=1, device_id=nbr,
            device_id_type=pl.DeviceIdType.MESH,
        )
    pl.semaphore_wait(barrier, 3)

    for i in range(2):
        xcopies[i].wait()

    @pl.when(my_y == 0)
    def _():
        for i in range(2):
            x_p[i * KH:(i + 1) * KH, :] = (
                x_stage[i][:, M_HALF:].astype(jnp.bfloat16))

    @pl.when(my_y == 1)
    def _():
        for i in range(2):
            x_p[i * KH:(i + 1) * KH, :] = (
                x_stage[i][:, :M_HALF].astype(jnp.bfloat16))

    dycopy.wait()

    def half_dot(xref, col_lo, width):
        return lax.dot_general(
            xref[:], dy_v[:, col_lo:col_lo + width].astype(jnp.bfloat16),
            (((0,), (0,)), ((), ())),
            preferred_element_type=jnp.float32,
        )

    y_send[0, :, :] = half_dot(x_p, 0, SC).astype(jnp.bfloat16)
    y_rdmas = [pltpu.make_async_remote_copy(
        src_ref=y_send.at[0], dst_ref=y_recv.at[0],
        send_sem=y_ssem.at[0], recv_sem=y_rsem.at[0],
        device_id=partner, device_id_type=pl.DeviceIdType.MESH,
    )]
    y_rdmas[0].start()

    @pl.when(my_y == 0)
    def _():
        for i in range(2):
            x_m[i * KH:(i + 1) * KH, :] = (
                x_stage[i][:, :M_HALF].astype(jnp.bfloat16))

    @pl.when(my_y == 1)
    def _():
        for i in range(2):
            x_m[i * KH:(i + 1) * KH, :] = (
                x_stage[i][:, M_HALF:].astype(jnp.bfloat16))

    red_bf[0, :, :] = half_dot(x_m, 0, SC).astype(jnp.bfloat16)

    for s in range(1, NSUB):
        y_send[s, :, :] = half_dot(x_p, s * SC, SC).astype(jnp.bfloat16)
        r = pltpu.make_async_remote_copy(
            src_ref=y_send.at[s], dst_ref=y_recv.at[s],
            send_sem=y_ssem.at[s], recv_sem=y_rsem.at[s],
            device_id=partner, device_id_type=pl.DeviceIdType.MESH,
        )
        r.start()
        y_rdmas.append(r)
        red_bf[s, :, :] = half_dot(x_m, s * SC, SC).astype(jnp.bfloat16)

    dy4copy = pltpu.make_async_copy(
        dy_hbm.at[:, pl.ds(p4 * FC, FC)], dy_v, load_sems.at[2]
    )
    dy4copy.start()

    store_jobs = []

    def store(col_start, width, src_ref):
        si = len(store_jobs)
        if si >= 4:
            store_jobs[si - 4].wait()
        st = pltpu.make_async_copy(
            src_ref,
            out_hbm.at[:, pl.ds(col_start, width)],
            store_sems.at[si % 4],
        )
        st.start()
        store_jobs.append(st)

    def hop_rdma(dirn, s, h):
        buf = cw_buf if dirn == "cw" else ccw_buf
        ssem = cw_ssem if dirn == "cw" else ccw_ssem
        rsem = cw_rsem if dirn == "cw" else ccw_rsem
        tgt = right if dirn == "cw" else left
        src = red_bf.at[s] if h == 0 else buf.at[s, h - 1]
        return pltpu.make_async_remote_copy(
            src_ref=src, dst_ref=buf.at[s, h],
            send_sem=ssem.at[s * HOPS + h], recv_sem=rsem.at[s * HOPS + h],
            device_id=tgt, device_id_type=pl.DeviceIdType.MESH,
        )

    rdmas = []
    live = {}
    for s in range(NSUB):
        y_rdmas[s].wait()
        red_bf[s, :, :] = (
            red_bf[s].astype(jnp.float32) + y_recv[s].astype(jnp.float32)
        ).astype(jnp.bfloat16)
        for dirn in ("cw", "ccw"):
            r = hop_rdma(dirn, s, 0)
            r.start()
            rdmas.append(r)
            live[(dirn, s)] = r
        store(p * FC + s * SC, SC, red_bf.at[s])

    y4_rdmas = []
    for h in range(HOPS):
        for s in range(NSUB):
            for dirn in ("cw", "ccw"):
                live[(dirn, s)].wait_recv()
                if h + 1 < HOPS:
                    r = hop_rdma(dirn, s, h + 1)
                    r.start()
                    rdmas.append(r)
                    live[(dirn, s)] = r

        if h < 2:
            if h == 0:
                dy4copy.wait()
            y4_send[h, :, :] = half_dot(
                x_p, h * P4SC, P4SC).astype(jnp.bfloat16)
            r4 = pltpu.make_async_remote_copy(
                src_ref=y4_send.at[h], dst_ref=y4_recv.at[h],
                send_sem=y4_ssem.at[h], recv_sem=y4_rsem.at[h],
                device_id=partner, device_id_type=pl.DeviceIdType.MESH,
            )
            r4.start()
            y4_rdmas.append(r4)
            red4_bf[h, :, :] = half_dot(
                x_m, h * P4SC, P4SC).astype(jnp.bfloat16)

        for s in range(NSUB):
            store(((p - h - 1) % NRING) * FC + s * SC, SC, cw_buf.at[s, h])
            store(((p + h + 1) % NRING) * FC + s * SC, SC, ccw_buf.at[s, h])

    for h in range(2):
        y4_rdmas[h].wait()
        red4_bf[h, :, :] = (
            red4_bf[h].astype(jnp.float32) + y4_recv[h].astype(jnp.float32)
        ).astype(jnp.bfloat16)
        store(p4 * FC + h * P4SC, P4SC, red4_bf.at[h])

    for r in rdmas:
        r.wait_send()
    for st in store_jobs[-4:]:
        st.wait()


def kernel(x, dy):
    return pl.pallas_call(
        _body,
        in_specs=[
            pl.BlockSpec(memory_space=pl.ANY),
            pl.BlockSpec(memory_space=pl.ANY),
        ],
        out_specs=pl.BlockSpec(memory_space=pl.ANY),
        out_shape=jax.ShapeDtypeStruct((M_HALF, F), jnp.bfloat16),
        scratch_shapes=[
            pltpu.VMEM((K, M_HALF), jnp.bfloat16),
            pltpu.VMEM((K, M_HALF), jnp.bfloat16),
            pltpu.VMEM((2, KH, D), jnp.float32),
            pltpu.VMEM((K, FC), jnp.float32),
            pltpu.VMEM((NSUB, M_HALF, SC), jnp.bfloat16),
            pltpu.VMEM((NSUB, M_HALF, SC), jnp.bfloat16),
            pltpu.VMEM((NSUB, M_HALF, SC), jnp.bfloat16),
            pltpu.VMEM((NSUB, HOPS, M_HALF, SC), jnp.bfloat16),
            pltpu.VMEM((NSUB, HOPS, M_HALF, SC), jnp.bfloat16),
            pltpu.VMEM((2, M_HALF, P4SC), jnp.bfloat16),
            pltpu.VMEM((2, M_HALF, P4SC), jnp.bfloat16),
            pltpu.VMEM((2, M_HALF, P4SC), jnp.bfloat16),
            pltpu.SemaphoreType.DMA((3,)),
            pltpu.SemaphoreType.DMA((4,)),
            pltpu.SemaphoreType.DMA((NSUB,)),
            pltpu.SemaphoreType.DMA((NSUB,)),
            pltpu.SemaphoreType.DMA((2,)),
            pltpu.SemaphoreType.DMA((2,)),
            pltpu.SemaphoreType.DMA((NSUB * HOPS,)),
            pltpu.SemaphoreType.DMA((NSUB * HOPS,)),
            pltpu.SemaphoreType.DMA((NSUB * HOPS,)),
            pltpu.SemaphoreType.DMA((NSUB * HOPS,)),
        ],
        compiler_params=pltpu.CompilerParams(
            collective_id=0,
            vmem_limit_bytes=62 * 1024 * 1024,
        ),
    )(x, dy)
